# baseline (speedup 1.0000x reference)
import jax
import jax.numpy as jnp
from jax.experimental import pallas as pl
from jax.experimental.pallas import tpu as pltpu

B, L, T, KH, DH = 16, 2, 2048, 8, 64
ROW = 2 * KH * DH
NC = 8
CM = (B * L) // NC


def _body(layer_ref, idx_ref, kv_ref, buf_any, out_any, copy_sem, row_sem):
    layer = layer_ref[0]
    idx = idx_ref[0]
    for c in range(NC):
        pltpu.make_async_copy(
            buf_any.at[pl.ds(c * CM, CM)], out_any.at[pl.ds(c * CM, CM)], copy_sem
        ).start()
    for c in range(NC):
        pltpu.make_async_copy(
            buf_any.at[pl.ds(c * CM, CM)], out_any.at[pl.ds(c * CM, CM)], copy_sem
        ).wait()
    for b in range(B):
        pltpu.make_async_copy(
            kv_ref.at[b], out_any.at[b * L + layer, idx], row_sem
        ).start()
    for b in range(B):
        pltpu.make_async_copy(
            kv_ref.at[b], out_any.at[b * L + layer, idx], row_sem
        ).wait()


@jax.jit
def _run(layer_s, idx_s, kv, buf3):
    return pl.pallas_call(
        _body,
        in_specs=[
            pl.BlockSpec(memory_space=pltpu.SMEM),
            pl.BlockSpec(memory_space=pltpu.SMEM),
            pl.BlockSpec(memory_space=pltpu.VMEM),
            pl.BlockSpec(memory_space=pl.ANY),
        ],
        out_specs=pl.BlockSpec(memory_space=pl.ANY),
        out_shape=jax.ShapeDtypeStruct((B * L, T, ROW), jnp.float32),
        scratch_shapes=[pltpu.SemaphoreType.DMA, pltpu.SemaphoreType.DMA],
    )(layer_s, idx_s, kv, buf3)


def kernel(buf, k_step, v_step, layer, idx):
    layer = jnp.clip(jnp.asarray(layer, jnp.int32), 0, L - 1)
    idx = jnp.clip(jnp.asarray(idx, jnp.int32), 0, T - 1)
    step = jnp.clip(idx, 0, k_step.shape[1] - 1)
    ks = jax.lax.dynamic_index_in_dim(k_step, step, axis=1, keepdims=False)
    vs = jax.lax.dynamic_index_in_dim(v_step, step, axis=1, keepdims=False)
    kv = jnp.concatenate([ks.reshape(B, KH * DH), vs.reshape(B, KH * DH)], axis=1)
    out3 = _run(layer.reshape(1), idx.reshape(1), kv, buf.reshape(B * L, T, ROW))
    return out3.reshape(B, L, T, 2, KH, DH)


# speedup vs baseline: 12.6593x; 12.6593x over previous
import jax
import jax.numpy as jnp
from jax.experimental import pallas as pl
from jax.experimental.pallas import tpu as pltpu

B, L, T, KH, DH = 16, 2, 2048, 8, 64
ROW = 2 * KH * DH
M = B * L
NBUF = 4


def _body(layer_ref, idx_ref, kv_ref, buf_any, out_any, vbuf, in_sems, out_sems, row_sem):
    layer = layer_ref[0]
    idx = idx_ref[0]

    def in_dma(i, s):
        return pltpu.make_async_copy(buf_any.at[pl.ds(i, 1)], vbuf.at[s], in_sems.at[s])

    def out_dma(i, s):
        return pltpu.make_async_copy(vbuf.at[s], out_any.at[pl.ds(i, 1)], out_sems.at[s])

    for i in range(M):
        s = i % NBUF
        if i >= NBUF:
            out_dma(i - NBUF, s).wait()
        in_dma(i, s).start()
        in_dma(i, s).wait()
        out_dma(i, s).start()
    for i in range(M - NBUF, M):
        out_dma(i, i % NBUF).wait()

    for b in range(B):
        pltpu.make_async_copy(
            kv_ref.at[b], out_any.at[b * L + layer, idx], row_sem
        ).start()
    for b in range(B):
        pltpu.make_async_copy(
            kv_ref.at[b], out_any.at[b * L + layer, idx], row_sem
        ).wait()


@jax.jit
def _run(layer_s, idx_s, kv, buf3):
    return pl.pallas_call(
        _body,
        in_specs=[
            pl.BlockSpec(memory_space=pltpu.SMEM),
            pl.BlockSpec(memory_space=pltpu.SMEM),
            pl.BlockSpec(memory_space=pltpu.VMEM),
            pl.BlockSpec(memory_space=pl.ANY),
        ],
        out_specs=pl.BlockSpec(memory_space=pl.ANY),
        out_shape=jax.ShapeDtypeStruct((M, T, ROW), jnp.float32),
        scratch_shapes=[
            pltpu.VMEM((NBUF, 1, T, ROW), jnp.float32),
            pltpu.SemaphoreType.DMA((NBUF,)),
            pltpu.SemaphoreType.DMA((NBUF,)),
            pltpu.SemaphoreType.DMA,
        ],
    )(layer_s, idx_s, kv, buf3)


def kernel(buf, k_step, v_step, layer, idx):
    layer = jnp.clip(jnp.asarray(layer, jnp.int32), 0, L - 1)
    idx = jnp.clip(jnp.asarray(idx, jnp.int32), 0, T - 1)
    step = jnp.clip(idx, 0, k_step.shape[1] - 1)
    ks = jax.lax.dynamic_index_in_dim(k_step, step, axis=1, keepdims=False)
    vs = jax.lax.dynamic_index_in_dim(v_step, step, axis=1, keepdims=False)
    kv = jnp.concatenate([ks.reshape(B, KH * DH), vs.reshape(B, KH * DH)], axis=1)
    out3 = _run(layer.reshape(1), idx.reshape(1), kv, buf.reshape(M, T, ROW))
    return out3.reshape(B, L, T, 2, KH, DH)
